# baseline (speedup 1.0000x reference)
import functools

import jax
import jax.numpy as jnp
from jax import lax
from jax.experimental import pallas as pl
from jax.experimental.pallas import tpu as pltpu
from jax.experimental.pallas import tpu_sc as plsc

E = 8
TOPK = 2
D = 1024
DFF = 768
T = 2048
BT = 128
NPAD = TOPK * T + E * BT
NB = NPAD // BT
NC = 2
NS = 16
NW = NC * NS
TPW = T // NW


def _router_body(x_ref, gw_ref, logits_ref, s1_ref, s2_ref, w1_ref, w2_ref,
                 be_ref):
    x = x_ref[...]
    gw = gw_ref[...]
    logits = lax.dot_general(x, gw, (((1,), (1,)), ((), ())),
                             preferred_element_type=jnp.float32)
    logits_ref[...] = logits

    m = jnp.max(logits, axis=-1, keepdims=True)
    p = jnp.exp(logits - m)
    probs = p / jnp.sum(p, axis=-1, keepdims=True)
    ids = lax.broadcasted_iota(jnp.int32, (T, E), 1)
    m1 = jnp.max(probs, axis=-1, keepdims=True)
    i1 = jnp.min(jnp.where(probs == m1, ids, E), axis=-1, keepdims=True)
    probs2 = jnp.where(ids == i1, -1.0, probs)
    m2 = jnp.max(probs2, axis=-1, keepdims=True)
    i2 = jnp.min(jnp.where(probs2 == m2, ids, E), axis=-1, keepdims=True)
    denom = m1 + m2
    w1_ref[...] = m1 / denom
    w2_ref[...] = m2 / denom

    oh1 = (ids == i1).astype(jnp.float32)
    oh2 = (ids == i2).astype(jnp.float32)
    cnt = oh1 + oh2

    G = 16
    GSZ = T // G
    ig = lax.broadcasted_iota(jnp.int32, (GSZ, GSZ), 0)
    jg = lax.broadcasted_iota(jnp.int32, (GSZ, GSZ), 1)
    tri = (jg < ig).astype(jnp.bfloat16)
    parts = []
    gtots = []
    for g in range(G):
        sub = cnt[g * GSZ:(g + 1) * GSZ, :]
        parts.append(lax.dot_general(
            tri, sub.astype(jnp.bfloat16), (((1,), (0,)), ((), ())),
            preferred_element_type=jnp.float32))
        gtots.append(jnp.sum(sub, axis=0, keepdims=True))
    gt = jnp.concatenate(gtots, axis=0)
    i16 = lax.broadcasted_iota(jnp.int32, (G, G), 0)
    j16 = lax.broadcasted_iota(jnp.int32, (G, G), 1)
    tri16 = (j16 < i16).astype(jnp.float32)
    gpre = lax.dot_general(tri16, gt, (((1,), (0,)), ((), ())),
                           preferred_element_type=jnp.float32)
    P = jnp.concatenate(
        [parts[g] + gpre[g:g + 1, :] for g in range(G)], axis=0)
    tot = gpre[G - 1:G, :] + gt[G - 1:G, :]

    pcq = jnp.floor((tot + (BT - 1)) / BT)
    a8 = lax.broadcasted_iota(jnp.int32, (E, E), 0)
    b8 = lax.broadcasted_iota(jnp.int32, (E, E), 1)
    pcqb = jnp.broadcast_to(pcq, (E, E))
    padq_col = jnp.sum(jnp.where(b8 < a8, pcqb, 0.0), axis=1,
                       keepdims=True)
    incl_col = jnp.sum(jnp.where(b8 <= a8, pcqb, 0.0), axis=1,
                       keepdims=True)
    pad_off_col = padq_col * BT

    po1 = lax.dot_general(oh1, pad_off_col, (((1,), (0,)), ((), ())),
                          preferred_element_type=jnp.float32)
    po2 = lax.dot_general(oh2, pad_off_col, (((1,), (0,)), ((), ())),
                          preferred_element_type=jnp.float32)
    r1 = jnp.sum(oh1 * P, axis=-1, keepdims=True)
    r2 = jnp.sum(oh2 * P, axis=-1, keepdims=True)
    s1_ref[...] = (po1 + r1).astype(jnp.int32)
    s2_ref[...] = (po2 + r2).astype(jnp.int32)

    bb = lax.broadcasted_iota(jnp.int32, (E, NB + 8), 1).astype(jnp.float32)
    cmp = (bb >= incl_col).astype(jnp.int32)
    be = jnp.minimum(jnp.sum(cmp, axis=0, keepdims=True), E - 1)
    col = lax.broadcasted_iota(jnp.int32, (1, NB + 8), 1)
    used = jnp.sum(pcq, axis=1, keepdims=True).astype(jnp.int32)
    be_ref[...] = jnp.where(col == NB, used, be)


def _router(x, gate_w):
    return pl.pallas_call(
        _router_body,
        out_shape=(
            jax.ShapeDtypeStruct((T, E), jnp.float32),
            jax.ShapeDtypeStruct((T, 1), jnp.int32),
            jax.ShapeDtypeStruct((T, 1), jnp.int32),
            jax.ShapeDtypeStruct((T, 1), jnp.float32),
            jax.ShapeDtypeStruct((T, 1), jnp.float32),
            jax.ShapeDtypeStruct((1, NB + 8), jnp.int32),
        ),
    )(x, gate_w)


def _dispatch_body(x_hbm, s1_hbm, s2_hbm, xs_hbm, s1_v, s2_v, rows_v, sem):
    wid = lax.axis_index("s") * NC + lax.axis_index("c")
    base = wid * TPW
    pltpu.sync_copy(s1_hbm.at[pl.ds(base, TPW)], s1_v)
    pltpu.sync_copy(s2_hbm.at[pl.ds(base, TPW)], s2_v)
    pltpu.sync_copy(x_hbm.at[pl.ds(base, TPW), :], rows_v)
    c1 = pltpu.async_copy(rows_v, xs_hbm.at[s1_v], sem)
    c2 = pltpu.async_copy(rows_v, xs_hbm.at[s2_v], sem)
    c1.wait()
    c2.wait()


def _dispatch(x, s1, s2):
    return pl.kernel(
        _dispatch_body,
        out_type=jax.ShapeDtypeStruct((NPAD, D), jnp.float32),
        mesh=plsc.VectorSubcoreMesh(core_axis_name="c", subcore_axis_name="s"),
        scratch_types=[
            pltpu.VMEM((TPW,), jnp.int32),
            pltpu.VMEM((TPW,), jnp.int32),
            pltpu.VMEM((TPW, D), jnp.float32),
            pltpu.SemaphoreType.DMA,
        ],
    )(x, s1, s2)


def _gmm_body(be_ref, xs_ref, w13_ref, w2_ref, out_ref):
    b = pl.program_id(0)

    @pl.when(b < be_ref[NB])
    def _():
        xb = xs_ref[...].astype(jnp.bfloat16)
        h = lax.dot_general(xb, w13_ref[0].astype(jnp.bfloat16),
                            (((1,), (1,)), ((), ())),
                            preferred_element_type=jnp.float32)
        g = h[:, :DFF]
        u = h[:, DFF:]
        a = (g / (1.0 + jnp.exp(-g))) * u
        y = lax.dot_general(a.astype(jnp.bfloat16),
                            w2_ref[0].astype(jnp.bfloat16),
                            (((1,), (1,)), ((), ())),
                            preferred_element_type=jnp.float32)
        out_ref[...] = y


def _gmm(be, xs, w13, w2):
    grid_spec = pltpu.PrefetchScalarGridSpec(
        num_scalar_prefetch=1,
        grid=(NB,),
        in_specs=[
            pl.BlockSpec((BT, D), lambda b, be: (b, 0)),
            pl.BlockSpec((1, 2 * DFF, D), lambda b, be: (be[b], 0, 0)),
            pl.BlockSpec((1, D, DFF), lambda b, be: (be[b], 0, 0)),
        ],
        out_specs=pl.BlockSpec((BT, D), lambda b, be: (b, 0)),
    )
    return pl.pallas_call(
        _gmm_body,
        grid_spec=grid_spec,
        out_shape=jax.ShapeDtypeStruct((NPAD, D), jnp.float32),
        compiler_params=pltpu.CompilerParams(
            dimension_semantics=("parallel",)),
    )(be, xs, w13, w2)


def _combine_body(ysw_hbm, s1_hbm, s2_hbm, yall_hbm, sv, buf, sem):
    wid = lax.axis_index("s") * NC + lax.axis_index("c")
    base = wid * TPW
    pltpu.sync_copy(s1_hbm.at[pl.ds(base, TPW)], sv)
    pltpu.async_copy(ysw_hbm.at[sv], buf, sem).wait()
    pltpu.sync_copy(buf, yall_hbm.at[pl.ds(base, TPW), :])
    pltpu.sync_copy(s2_hbm.at[pl.ds(base, TPW)], sv)
    pltpu.async_copy(ysw_hbm.at[sv], buf, sem).wait()
    pltpu.sync_copy(buf, yall_hbm.at[pl.ds(T + base, TPW), :])


def _combine(ysw, s1, s2):
    return pl.kernel(
        _combine_body,
        out_type=jax.ShapeDtypeStruct((2 * T, D), jnp.float32),
        mesh=plsc.VectorSubcoreMesh(core_axis_name="c", subcore_axis_name="s"),
        scratch_types=[
            pltpu.VMEM((TPW,), jnp.int32),
            pltpu.VMEM((TPW, D), jnp.float32),
            pltpu.SemaphoreType.DMA,
        ],
    )(ysw, s1, s2)


def _add_body(y1_ref, y2_ref, w1_ref, w2_ref, o_ref):
    o_ref[...] = y1_ref[...] * w1_ref[...] + y2_ref[...] * w2_ref[...]


def _wadd(yall, w1, w2):
    nb = T // BT
    return pl.pallas_call(
        _add_body,
        grid=(nb,),
        in_specs=[
            pl.BlockSpec((BT, D), lambda b: (b, 0)),
            pl.BlockSpec((BT, D), lambda b: (b + nb, 0)),
            pl.BlockSpec((BT, 1), lambda b: (b, 0)),
            pl.BlockSpec((BT, 1), lambda b: (b, 0)),
        ],
        out_specs=pl.BlockSpec((BT, D), lambda b: (b, 0)),
        out_shape=jax.ShapeDtypeStruct((T, D), jnp.float32),
        compiler_params=pltpu.CompilerParams(
            dimension_semantics=("parallel",)),
    )(yall, yall, w1, w2)


@jax.jit
def kernel(hidden_states, gate_w, w13, w2):
    x = hidden_states.reshape(T, D)
    logits, s1, s2, w1, w2g, be = _router(x, gate_w)
    s1 = s1.reshape(T)
    s2 = s2.reshape(T)
    xs = _dispatch(x, s1, s2)
    ysw = _gmm(be.reshape(NB + 8), xs, w13, w2)
    yall = _combine(ysw, s1, s2)
    out = _wadd(yall, w1, w2g)
    return out, logits


# speedup vs baseline: 1.0027x; 1.0027x over previous
import functools

import jax
import jax.numpy as jnp
from jax import lax
from jax.experimental import pallas as pl
from jax.experimental.pallas import tpu as pltpu
from jax.experimental.pallas import tpu_sc as plsc

E = 8
TOPK = 2
D = 1024
DFF = 768
T = 2048
BT = 128
NPAD = TOPK * T + E * BT
NB = NPAD // BT
NC = 2
NS = 16
NW = NC * NS
TPW = T // NW


def _router_body(x_ref, gw_ref, logits_ref, s1_ref, s2_ref, w1_ref, w2_ref,
                 be_ref):
    x = x_ref[...]
    gw = gw_ref[...]
    logits = lax.dot_general(x, gw, (((1,), (1,)), ((), ())),
                             preferred_element_type=jnp.float32)
    logits_ref[...] = logits

    m = jnp.max(logits, axis=-1, keepdims=True)
    p = jnp.exp(logits - m)
    probs = p / jnp.sum(p, axis=-1, keepdims=True)
    ids = lax.broadcasted_iota(jnp.int32, (T, E), 1)
    m1 = jnp.max(probs, axis=-1, keepdims=True)
    i1 = jnp.min(jnp.where(probs == m1, ids, E), axis=-1, keepdims=True)
    probs2 = jnp.where(ids == i1, -1.0, probs)
    m2 = jnp.max(probs2, axis=-1, keepdims=True)
    i2 = jnp.min(jnp.where(probs2 == m2, ids, E), axis=-1, keepdims=True)
    denom = m1 + m2
    w1_ref[...] = m1 / denom
    w2_ref[...] = m2 / denom

    oh1 = (ids == i1).astype(jnp.float32)
    oh2 = (ids == i2).astype(jnp.float32)
    cnt = oh1 + oh2

    G = 16
    GSZ = T // G
    ig = lax.broadcasted_iota(jnp.int32, (GSZ, GSZ), 0)
    jg = lax.broadcasted_iota(jnp.int32, (GSZ, GSZ), 1)
    tri = (jg < ig).astype(jnp.bfloat16)
    parts = []
    gtots = []
    for g in range(G):
        sub = cnt[g * GSZ:(g + 1) * GSZ, :]
        parts.append(lax.dot_general(
            tri, sub.astype(jnp.bfloat16), (((1,), (0,)), ((), ())),
            preferred_element_type=jnp.float32))
        gtots.append(jnp.sum(sub, axis=0, keepdims=True))
    gt = jnp.concatenate(gtots, axis=0)
    i16 = lax.broadcasted_iota(jnp.int32, (G, G), 0)
    j16 = lax.broadcasted_iota(jnp.int32, (G, G), 1)
    tri16 = (j16 < i16).astype(jnp.float32)
    gpre = lax.dot_general(tri16, gt, (((1,), (0,)), ((), ())),
                           preferred_element_type=jnp.float32)
    P = jnp.concatenate(
        [parts[g] + gpre[g:g + 1, :] for g in range(G)], axis=0)
    tot = gpre[G - 1:G, :] + gt[G - 1:G, :]

    pcq = jnp.floor((tot + (BT - 1)) / BT)
    a8 = lax.broadcasted_iota(jnp.int32, (E, E), 0)
    b8 = lax.broadcasted_iota(jnp.int32, (E, E), 1)
    pcqb = jnp.broadcast_to(pcq, (E, E))
    padq_col = jnp.sum(jnp.where(b8 < a8, pcqb, 0.0), axis=1,
                       keepdims=True)
    incl_col = jnp.sum(jnp.where(b8 <= a8, pcqb, 0.0), axis=1,
                       keepdims=True)
    pad_off_col = padq_col * BT

    po1 = lax.dot_general(oh1, pad_off_col, (((1,), (0,)), ((), ())),
                          preferred_element_type=jnp.float32)
    po2 = lax.dot_general(oh2, pad_off_col, (((1,), (0,)), ((), ())),
                          preferred_element_type=jnp.float32)
    r1 = jnp.sum(oh1 * P, axis=-1, keepdims=True)
    r2 = jnp.sum(oh2 * P, axis=-1, keepdims=True)
    s1_ref[...] = (po1 + r1).astype(jnp.int32)
    s2_ref[...] = (po2 + r2).astype(jnp.int32)

    bb = lax.broadcasted_iota(jnp.int32, (E, NB + 8), 1).astype(jnp.float32)
    cmp = (bb >= incl_col).astype(jnp.int32)
    be = jnp.minimum(jnp.sum(cmp, axis=0, keepdims=True), E - 1)
    col = lax.broadcasted_iota(jnp.int32, (1, NB + 8), 1)
    used = jnp.sum(pcq, axis=1, keepdims=True).astype(jnp.int32)
    be_ref[...] = jnp.where(col == NB, used, be)


def _router(x, gate_w):
    return pl.pallas_call(
        _router_body,
        out_shape=(
            jax.ShapeDtypeStruct((T, E), jnp.float32),
            jax.ShapeDtypeStruct((T, 1), jnp.int32),
            jax.ShapeDtypeStruct((T, 1), jnp.int32),
            jax.ShapeDtypeStruct((T, 1), jnp.float32),
            jax.ShapeDtypeStruct((T, 1), jnp.float32),
            jax.ShapeDtypeStruct((1, NB + 8), jnp.int32),
        ),
    )(x, gate_w)


def _dispatch_body(x_hbm, s1_hbm, s2_hbm, xs_hbm, s1_v, s2_v, rows_v, sem):
    wid = lax.axis_index("s") * NC + lax.axis_index("c")
    base = wid * TPW
    pltpu.sync_copy(s1_hbm.at[pl.ds(base, TPW)], s1_v)
    pltpu.sync_copy(s2_hbm.at[pl.ds(base, TPW)], s2_v)
    pltpu.sync_copy(x_hbm.at[pl.ds(base, TPW), :], rows_v)
    c1 = pltpu.async_copy(rows_v, xs_hbm.at[s1_v], sem)
    c2 = pltpu.async_copy(rows_v, xs_hbm.at[s2_v], sem)
    c1.wait()
    c2.wait()


def _dispatch(x, s1, s2):
    return pl.kernel(
        _dispatch_body,
        out_type=jax.ShapeDtypeStruct((NPAD, D), jnp.float32),
        mesh=plsc.VectorSubcoreMesh(core_axis_name="c", subcore_axis_name="s"),
        scratch_types=[
            pltpu.VMEM((TPW,), jnp.int32),
            pltpu.VMEM((TPW,), jnp.int32),
            pltpu.VMEM((TPW, D), jnp.float32),
            pltpu.SemaphoreType.DMA,
        ],
    )(x, s1, s2)


def _gmm_body(be_ref, xs_ref, w13_ref, w2_ref, out_ref):
    b = pl.program_id(0)

    @pl.when(b < be_ref[NB])
    def _():
        xb = xs_ref[...].astype(jnp.bfloat16)
        h = lax.dot_general(xb, w13_ref[0].astype(jnp.bfloat16),
                            (((1,), (1,)), ((), ())),
                            preferred_element_type=jnp.float32)
        g = h[:, :DFF]
        u = h[:, DFF:]
        a = (g / (1.0 + jnp.exp(-g))) * u
        y = lax.dot_general(a.astype(jnp.bfloat16),
                            w2_ref[0].astype(jnp.bfloat16),
                            (((1,), (1,)), ((), ())),
                            preferred_element_type=jnp.float32)
        out_ref[...] = y


def _gmm(be, xs, w13, w2):
    grid_spec = pltpu.PrefetchScalarGridSpec(
        num_scalar_prefetch=1,
        grid=(NB,),
        in_specs=[
            pl.BlockSpec((BT, D), lambda b, be: (b, 0)),
            pl.BlockSpec((1, 2 * DFF, D), lambda b, be: (be[b], 0, 0)),
            pl.BlockSpec((1, D, DFF), lambda b, be: (be[b], 0, 0)),
        ],
        out_specs=pl.BlockSpec((BT, D), lambda b, be: (b, 0)),
    )
    return pl.pallas_call(
        _gmm_body,
        grid_spec=grid_spec,
        out_shape=jax.ShapeDtypeStruct((NPAD, D), jnp.float32),
    )(be, xs, w13, w2)


def _combine_body(ysw_hbm, s1_hbm, s2_hbm, yall_hbm, sv, buf, sem):
    wid = lax.axis_index("s") * NC + lax.axis_index("c")
    base = wid * TPW
    pltpu.sync_copy(s1_hbm.at[pl.ds(base, TPW)], sv)
    pltpu.async_copy(ysw_hbm.at[sv], buf, sem).wait()
    pltpu.sync_copy(buf, yall_hbm.at[pl.ds(base, TPW), :])
    pltpu.sync_copy(s2_hbm.at[pl.ds(base, TPW)], sv)
    pltpu.async_copy(ysw_hbm.at[sv], buf, sem).wait()
    pltpu.sync_copy(buf, yall_hbm.at[pl.ds(T + base, TPW), :])


def _combine(ysw, s1, s2):
    return pl.kernel(
        _combine_body,
        out_type=jax.ShapeDtypeStruct((2 * T, D), jnp.float32),
        mesh=plsc.VectorSubcoreMesh(core_axis_name="c", subcore_axis_name="s"),
        scratch_types=[
            pltpu.VMEM((TPW,), jnp.int32),
            pltpu.VMEM((TPW, D), jnp.float32),
            pltpu.SemaphoreType.DMA,
        ],
    )(ysw, s1, s2)


def _add_body(y1_ref, y2_ref, w1_ref, w2_ref, o_ref):
    o_ref[...] = y1_ref[...] * w1_ref[...] + y2_ref[...] * w2_ref[...]


def _wadd(yall, w1, w2):
    nb = T // BT
    return pl.pallas_call(
        _add_body,
        grid=(nb,),
        in_specs=[
            pl.BlockSpec((BT, D), lambda b: (b, 0)),
            pl.BlockSpec((BT, D), lambda b: (b + nb, 0)),
            pl.BlockSpec((BT, 1), lambda b: (b, 0)),
            pl.BlockSpec((BT, 1), lambda b: (b, 0)),
        ],
        out_specs=pl.BlockSpec((BT, D), lambda b: (b, 0)),
        out_shape=jax.ShapeDtypeStruct((T, D), jnp.float32),
    )(yall, yall, w1, w2)


@jax.jit
def kernel(hidden_states, gate_w, w13, w2):
    x = hidden_states.reshape(T, D)
    logits, s1, s2, w1, w2g, be = _router(x, gate_w)
    s1 = s1.reshape(T)
    s2 = s2.reshape(T)
    xs = _dispatch(x, s1, s2)
    ysw = _gmm(be.reshape(NB + 8), xs, w13, w2)
    yall = _combine(ysw, s1, s2)
    out = _wadd(yall, w1, w2g)
    return out, logits
